# baseline (speedup 1.0000x reference)
import functools

import jax
import jax.numpy as jnp
from jax import lax
from jax.experimental import pallas as pl
from jax.experimental.pallas import tpu as pltpu
from jax.experimental.pallas import tpu_sc as plsc

B, S = 4096, 50
SP = 56
D = 256
O = 64
TP = B * SP
NC, NS = 2, 16
NW = NC * NS
CHUNK = 128
T_PER_W = TP // NW
NCHUNK = T_PER_W // CHUNK


@functools.partial(
    pl.kernel,
    out_type=jax.ShapeDtypeStruct((TP, D // 2), jnp.uint32),
    mesh=plsc.VectorSubcoreMesh(core_axis_name="c", subcore_axis_name="s"),
    scratch_types=[
        pltpu.VMEM((T_PER_W,), jnp.int32),
        pltpu.VMEM((CHUNK, D), jnp.uint32),
        pltpu.VMEM((CHUNK, D), jnp.uint32),
        pltpu.VMEM((CHUNK, D // 2), jnp.uint32),
        pltpu.VMEM((CHUNK, D // 2), jnp.uint32),
        pltpu.SemaphoreType.DMA,
        pltpu.SemaphoreType.DMA,
        pltpu.SemaphoreType.DMA,
        pltpu.SemaphoreType.DMA,
    ],
)
def _sc_gather(table_hbm, idx_hbm, h_hbm, idx_v, buf_a, buf_b, pk_a, pk_b,
               gsem_a, gsem_b, osem_a, osem_b):
    wid = lax.axis_index("s") * NC + lax.axis_index("c")
    base = wid * T_PER_W
    pltpu.sync_copy(idx_hbm.at[pl.ds(base, T_PER_W)], idx_v)

    def start_gather(c, buf, gsem):
        pltpu.async_copy(
            table_hbm.at[idx_v.at[pl.ds(c * CHUNK, CHUNK)]], buf, gsem)

    def wait_gather(c, buf, gsem):
        pltpu.make_async_copy(
            table_hbm.at[idx_v.at[pl.ds(c * CHUNK, CHUNK)]], buf, gsem).wait()

    def convert(buf, pk):
        rnd = jnp.uint32(0x8000)
        himask = jnp.uint32(0xFFFF0000)

        def cbody(j, carry):
            for k in range(D // 32):
                lo = buf[j, pl.ds(32 * k, 16)]
                hi = buf[j, pl.ds(32 * k + 16, 16)]
                lo_bits = jnp.right_shift(lo + rnd, jnp.uint32(16))
                hi_bits = jnp.bitwise_and(hi + rnd, himask)
                word = jnp.bitwise_or(lo_bits, hi_bits)
                pk[j, pl.ds(16 * k, 16)] = word
            return carry
        lax.fori_loop(0, CHUNK, cbody, 0)

    def start_out(c, pk, osem):
        pltpu.async_copy(pk, h_hbm.at[pl.ds(base + c * CHUNK, CHUNK)], osem)

    def wait_out(c, pk, osem):
        pltpu.make_async_copy(
            pk, h_hbm.at[pl.ds(base + c * CHUNK, CHUNK)], osem).wait()

    start_gather(0, buf_a, gsem_a)

    def half(k, c, buf, pk, gsem, osem, buf_n, gsem_n, last):
        wait_gather(c, buf, gsem)

        @pl.when(c + 1 < NCHUNK)
        def _():
            start_gather(c + 1, buf_n, gsem_n)
        convert(buf, pk)

        @pl.when(k > 0)
        def _():
            wait_out(c - 2, pk, osem)
        start_out(c, pk, osem)

    def body(k, carry):
        half(k, 2 * k, buf_a, pk_a, gsem_a, osem_a, buf_b, gsem_b, False)
        half(k, 2 * k + 1, buf_b, pk_b, gsem_b, osem_b, buf_a, gsem_a, True)
        return carry

    lax.fori_loop(0, NCHUNK // 2, body, 0)
    wait_out(NCHUNK - 2, pk_a, osem_a)
    wait_out(NCHUNK - 1, pk_b, osem_b)


BLK_B = 64
BLK_T = BLK_B * SP


def _tc_matmul_body(h_ref, wlo_ref, whi_ref, b_ref, o_ref):
    hw = h_ref[...]
    lo = lax.bitcast_convert_type(
        lax.shift_left(hw, jnp.uint32(16)), jnp.float32)
    hi = lax.bitcast_convert_type(
        jnp.bitwise_and(hw, jnp.uint32(0xFFFF0000)), jnp.float32)
    dn = (((1,), (1,)), ((), ()))
    acc = (
        lax.dot_general(lo, wlo_ref[...], dn, preferred_element_type=jnp.float32)
        + lax.dot_general(hi, whi_ref[...], dn, preferred_element_type=jnp.float32)
        + b_ref[...]
    )
    o_ref[...] = acc.reshape(BLK_B, SP, O)


def _tc_matmul(h, Wlo, Whi, b):
    return pl.pallas_call(
        _tc_matmul_body,
        grid=(B // BLK_B,),
        in_specs=[
            pl.BlockSpec((BLK_T, D // 2), lambda i: (i, 0)),
            pl.BlockSpec((O, D // 2), lambda i: (0, 0)),
            pl.BlockSpec((O, D // 2), lambda i: (0, 0)),
            pl.BlockSpec((1, O), lambda i: (0, 0)),
        ],
        out_specs=pl.BlockSpec((BLK_B, SP, O), lambda i: (i, 0, 0)),
        out_shape=jax.ShapeDtypeStruct((B, SP, O), jnp.float32),
    )(h, Wlo, Whi, b.reshape(1, O))


def kernel(x, table, W, b):
    xp = jnp.pad(x.astype(jnp.int32), ((0, 0), (0, SP - S)),
                 mode="edge").reshape(TP)
    j = jnp.arange(D // 2)
    k, i = j // 16, j % 16
    Wlo = W[:, 32 * k + i]
    Whi = W[:, 32 * k + 16 + i]
    table_u = lax.bitcast_convert_type(table, jnp.uint32)
    h = _sc_gather(table_u, xp)
    return _tc_matmul(h, Wlo, Whi, b)[:, :S, :]
